# baseline (device time: 19095 ns/iter reference)
import jax
import jax.numpy as jnp
from jax import lax
from jax.experimental import pallas as pl
from jax.experimental.pallas import tpu as pltpu

N_DEV = 4


def kernel(x, w_mat):
    m, k_per = x.shape
    _, n = w_mat.shape
    blk = m // N_DEV

    def body(x_ref, w_ref, out_ref, xq, sc_tile, recvq, sc_recv,
             wf32, wbf, send_sems, recv_sems, sc_send_sems, sc_recv_sems,
             wdma_sems):
        my = lax.axis_index("i")

        offs = (0, 3, 1, 2)

        def wdma(j):
            kidx = lax.rem(my + offs[j], N_DEV)
            return pltpu.make_async_copy(
                w_ref.at[pl.ds(kidx * blk, blk), :],
                wf32.at[j % 2],
                wdma_sems.at[j],
            )

        barrier = pltpu.get_barrier_semaphore()
        for off in range(1, N_DEV):
            peer = lax.rem(my + off, N_DEV)
            pl.semaphore_signal(
                barrier, inc=1, device_id=(peer,),
                device_id_type=pl.DeviceIdType.MESH,
            )

        wdma(0).start()
        wdma(1).start()

        def quantize(off):
            peer = lax.rem(my + off, N_DEV)
            b = x_ref[pl.ds(peer * blk, blk), :]
            absmax = jnp.maximum(jnp.max(jnp.abs(b)), 1e-30)
            sc_tile[off - 1, :, :] = jnp.full((8, 128), absmax / 127.0,
                                              jnp.float32)
            xq[pl.ds(peer * blk, blk), :] = jnp.round(
                b * (127.0 / absmax)
            ).astype(jnp.int8)

        def send(off):
            peer = lax.rem(my + off, N_DEV)
            sc_rdma = pltpu.make_async_remote_copy(
                src_ref=sc_tile.at[off - 1],
                dst_ref=sc_recv.at[off - 1],
                send_sem=sc_send_sems.at[off - 1],
                recv_sem=sc_recv_sems.at[off - 1],
                device_id=(peer,),
                device_id_type=pl.DeviceIdType.MESH,
            )
            sc_rdma.start()
            rdma = pltpu.make_async_remote_copy(
                src_ref=xq.at[pl.ds(peer * blk, blk), :],
                dst_ref=recvq.at[off - 1],
                send_sem=send_sems.at[off - 1],
                recv_sem=recv_sems.at[off - 1],
                device_id=(peer,),
                device_id_type=pl.DeviceIdType.MESH,
            )
            rdma.start()
            return [sc_rdma, rdma]

        quantize(1)
        pl.semaphore_wait(barrier, N_DEV - 1)
        sends = send(1)
        quantize(3)
        sends += send(3)
        quantize(2)
        sends += send(2)

        wdma(0).wait()
        wbf[0, :, :] = wf32[0].astype(jnp.bfloat16)
        wdma(2).start()
        acc = jnp.dot(
            x_ref[pl.ds(my * blk, blk), :].astype(jnp.bfloat16), wbf[0],
            preferred_element_type=jnp.float32,
        )

        def wait_and_dequant(j, slot):
            s = j % 2
            wdma(j).wait()
            wbf[s, :, :] = wf32[s].astype(jnp.bfloat16)
            if j + 2 < N_DEV:
                wdma(j + 2).start()
            src = lax.rem(my - (slot + 1) + N_DEV, N_DEV)
            sc_wait = pltpu.make_async_remote_copy(
                src_ref=sc_recv.at[slot],
                dst_ref=sc_recv.at[slot],
                send_sem=sc_send_sems.at[slot],
                recv_sem=sc_recv_sems.at[slot],
                device_id=(src,),
                device_id_type=pl.DeviceIdType.MESH,
            )
            sc_wait.wait_recv()
            recv = pltpu.make_async_remote_copy(
                src_ref=recvq.at[slot],
                dst_ref=recvq.at[slot],
                send_sem=send_sems.at[slot],
                recv_sem=recv_sems.at[slot],
                device_id=(src,),
                device_id_type=pl.DeviceIdType.MESH,
            )
            recv.wait_recv()
            return (
                recvq[slot].astype(jnp.float32) * sc_recv[slot, 0, 0]
            ).astype(jnp.bfloat16)

        def gelu(v):
            c = 0.7978845608028654
            return 0.5 * v * (1.0 + jnp.tanh(c * (v + 0.044715 * v * v * v)))

        for j, slot in ((1, 0), (2, 2)):
            xhat = wait_and_dequant(j, slot)
            acc = acc + jnp.dot(
                xhat, wbf[j % 2],
                preferred_element_type=jnp.float32,
            )

        xhat = wait_and_dequant(3, 1)
        h = n // 2
        s = 3 % 2
        out_ref[:, 0:h] = gelu(
            acc[:, 0:h] + jnp.dot(
                xhat, wbf[s, :, 0:h], preferred_element_type=jnp.float32
            )
        )
        out_ref[:, h:] = gelu(
            acc[:, h:] + jnp.dot(
                xhat, wbf[s, :, h:], preferred_element_type=jnp.float32
            )
        )

        for rdma in sends:
            rdma.wait_send()

    return pl.pallas_call(
        body,
        out_shape=jax.ShapeDtypeStruct((blk, n), jnp.float32),
        in_specs=[
            pl.BlockSpec(memory_space=pltpu.VMEM),
            pl.BlockSpec(memory_space=pl.ANY),
        ],
        out_specs=pl.BlockSpec(memory_space=pltpu.VMEM),
        scratch_shapes=[
            pltpu.VMEM((m, k_per), jnp.int8),
            pltpu.VMEM((N_DEV - 1, 8, 128), jnp.float32),
            pltpu.VMEM((N_DEV - 1, blk, k_per), jnp.int8),
            pltpu.VMEM((N_DEV - 1, 8, 128), jnp.float32),
            pltpu.VMEM((2, blk, n), jnp.float32),
            pltpu.VMEM((2, blk, n), jnp.bfloat16),
            pltpu.SemaphoreType.DMA((N_DEV - 1,)),
            pltpu.SemaphoreType.DMA((N_DEV - 1,)),
            pltpu.SemaphoreType.DMA((N_DEV - 1,)),
            pltpu.SemaphoreType.DMA((N_DEV - 1,)),
            pltpu.SemaphoreType.DMA((N_DEV,)),
        ],
        compiler_params=pltpu.CompilerParams(collective_id=0),
    )(x, w_mat)


# device time: 18984 ns/iter; 1.0058x vs baseline; 1.0058x over previous
import jax
import jax.numpy as jnp
from jax import lax
from jax.experimental import pallas as pl
from jax.experimental.pallas import tpu as pltpu

N_DEV = 4


def kernel(x, w_mat):
    m, k_per = x.shape
    _, n = w_mat.shape
    blk = m // N_DEV
    h = n // 2

    def body(x_ref, w_ref, out_ref, xq, sc_tile, recvq, sc_recv,
             wf32, wbf, send_sems, recv_sems, sc_send_sems, sc_recv_sems,
             wdma_sems):
        my = lax.axis_index("i")

        offs = (0, 3, 1, 2)

        def wdma(j):
            kidx = lax.rem(my + offs[j], N_DEV)
            return pltpu.make_async_copy(
                w_ref.at[pl.ds(kidx * blk, blk), :],
                wf32.at[j % 2],
                wdma_sems.at[j],
            )

        def wdma0_half(half):
            lo = half * h
            return pltpu.make_async_copy(
                w_ref.at[pl.ds(my * blk, blk), pl.ds(lo, h)],
                wf32.at[0, :, pl.ds(lo, h)],
                wdma_sems.at[0 if half == 0 else N_DEV],
            )

        barrier = pltpu.get_barrier_semaphore()
        for off in range(1, N_DEV):
            peer = lax.rem(my + off, N_DEV)
            pl.semaphore_signal(
                barrier, inc=1, device_id=(peer,),
                device_id_type=pl.DeviceIdType.MESH,
            )

        wdma0_half(0).start()
        wdma0_half(1).start()
        wdma(1).start()

        def quantize(off):
            peer = lax.rem(my + off, N_DEV)
            b = x_ref[pl.ds(peer * blk, blk), :]
            absmax = jnp.maximum(jnp.max(jnp.abs(b)), 1e-30)
            sc_tile[off - 1, :, :] = jnp.full((8, 128), absmax / 127.0,
                                              jnp.float32)
            xq[pl.ds(peer * blk, blk), :] = jnp.round(
                b * (127.0 / absmax)
            ).astype(jnp.int8)

        def send(off):
            peer = lax.rem(my + off, N_DEV)
            sc_rdma = pltpu.make_async_remote_copy(
                src_ref=sc_tile.at[off - 1],
                dst_ref=sc_recv.at[off - 1],
                send_sem=sc_send_sems.at[off - 1],
                recv_sem=sc_recv_sems.at[off - 1],
                device_id=(peer,),
                device_id_type=pl.DeviceIdType.MESH,
            )
            sc_rdma.start()
            rdma = pltpu.make_async_remote_copy(
                src_ref=xq.at[pl.ds(peer * blk, blk), :],
                dst_ref=recvq.at[off - 1],
                send_sem=send_sems.at[off - 1],
                recv_sem=recv_sems.at[off - 1],
                device_id=(peer,),
                device_id_type=pl.DeviceIdType.MESH,
            )
            rdma.start()
            return [sc_rdma, rdma]

        quantize(1)
        pl.semaphore_wait(barrier, N_DEV - 1)
        sends = send(1)
        quantize(3)
        sends += send(3)
        quantize(2)
        sends += send(2)

        xloc = x_ref[pl.ds(my * blk, blk), :].astype(jnp.bfloat16)
        wdma0_half(0).wait()
        wbf[0, :, 0:h] = wf32[0, :, 0:h].astype(jnp.bfloat16)
        acc_a = jnp.dot(
            xloc, wbf[0, :, 0:h], preferred_element_type=jnp.float32
        )
        wdma0_half(1).wait()
        wbf[0, :, h:] = wf32[0, :, h:].astype(jnp.bfloat16)
        wdma(2).start()
        acc_b = jnp.dot(
            xloc, wbf[0, :, h:], preferred_element_type=jnp.float32
        )

        def wait_and_prep(j, slot):
            s = j % 2
            src = lax.rem(my - (slot + 1) + N_DEV, N_DEV)
            sc_wait = pltpu.make_async_remote_copy(
                src_ref=sc_recv.at[slot],
                dst_ref=sc_recv.at[slot],
                send_sem=sc_send_sems.at[slot],
                recv_sem=sc_recv_sems.at[slot],
                device_id=(src,),
                device_id_type=pl.DeviceIdType.MESH,
            )
            sc_wait.wait_recv()
            wdma(j).wait()
            wbf[s, :, :] = (
                wf32[s] * sc_recv[slot, 0, 0]
            ).astype(jnp.bfloat16)
            if j + 2 < N_DEV:
                wdma(j + 2).start()
            recv = pltpu.make_async_remote_copy(
                src_ref=recvq.at[slot],
                dst_ref=recvq.at[slot],
                send_sem=send_sems.at[slot],
                recv_sem=recv_sems.at[slot],
                device_id=(src,),
                device_id_type=pl.DeviceIdType.MESH,
            )
            recv.wait_recv()
            return recvq[slot].astype(jnp.bfloat16)

        for j, slot in ((1, 0), (2, 2)):
            xhat = wait_and_prep(j, slot)
            s = j % 2
            acc_a = acc_a + jnp.dot(
                xhat, wbf[s, :, 0:h], preferred_element_type=jnp.float32
            )
            acc_b = acc_b + jnp.dot(
                xhat, wbf[s, :, h:], preferred_element_type=jnp.float32
            )

        def gelu(v):
            c = 0.7978845608028654
            return 0.5 * v * (1.0 + jnp.tanh(c * (v + 0.044715 * v * v * v)))

        xhat = wait_and_prep(3, 1)
        out_ref[:, 0:h] = gelu(
            acc_a + jnp.dot(
                xhat, wbf[1, :, 0:h], preferred_element_type=jnp.float32
            )
        )
        out_ref[:, h:] = gelu(
            acc_b + jnp.dot(
                xhat, wbf[1, :, h:], preferred_element_type=jnp.float32
            )
        )

        for rdma in sends:
            rdma.wait_send()

    return pl.pallas_call(
        body,
        out_shape=jax.ShapeDtypeStruct((blk, n), jnp.float32),
        in_specs=[
            pl.BlockSpec(memory_space=pltpu.VMEM),
            pl.BlockSpec(memory_space=pl.ANY),
        ],
        out_specs=pl.BlockSpec(memory_space=pltpu.VMEM),
        scratch_shapes=[
            pltpu.VMEM((m, k_per), jnp.int8),
            pltpu.VMEM((N_DEV - 1, 8, 128), jnp.float32),
            pltpu.VMEM((N_DEV - 1, blk, k_per), jnp.int8),
            pltpu.VMEM((N_DEV - 1, 8, 128), jnp.float32),
            pltpu.VMEM((2, blk, n), jnp.float32),
            pltpu.VMEM((2, blk, n), jnp.bfloat16),
            pltpu.SemaphoreType.DMA((N_DEV - 1,)),
            pltpu.SemaphoreType.DMA((N_DEV - 1,)),
            pltpu.SemaphoreType.DMA((N_DEV - 1,)),
            pltpu.SemaphoreType.DMA((N_DEV - 1,)),
            pltpu.SemaphoreType.DMA((N_DEV + 1,)),
        ],
        compiler_params=pltpu.CompilerParams(collective_id=0),
    )(x, w_mat)


# device time: 18537 ns/iter; 1.0301x vs baseline; 1.0241x over previous
import jax
import jax.numpy as jnp
from jax import lax
from jax.experimental import pallas as pl
from jax.experimental.pallas import tpu as pltpu

N_DEV = 4


def kernel(x, w_mat):
    m, k_per = x.shape
    _, n = w_mat.shape
    blk = m // N_DEV
    h = n // 2

    def body(x_ref, w_ref, out_ref, xq, sc_tile, recvq, sc_recv,
             wf32, send_sems, recv_sems, sc_send_sems, sc_recv_sems,
             wdma_sems):
        my = lax.axis_index("i")

        offs = (0, 3, 1, 2)
        bufs = (0, 1, 2, 0)

        def wdma(j):
            kidx = lax.rem(my + offs[j], N_DEV)
            return pltpu.make_async_copy(
                w_ref.at[pl.ds(kidx * blk, blk), :],
                wf32.at[bufs[j]],
                wdma_sems.at[j],
            )

        def wdma0_half(half):
            lo = half * h
            return pltpu.make_async_copy(
                w_ref.at[pl.ds(my * blk, blk), pl.ds(lo, h)],
                wf32.at[0, :, pl.ds(lo, h)],
                wdma_sems.at[0 if half == 0 else N_DEV],
            )

        barrier = pltpu.get_barrier_semaphore()
        for off in range(1, N_DEV):
            peer = lax.rem(my + off, N_DEV)
            pl.semaphore_signal(
                barrier, inc=1, device_id=(peer,),
                device_id_type=pl.DeviceIdType.MESH,
            )

        wdma0_half(0).start()
        wdma0_half(1).start()
        wdma(1).start()
        wdma(2).start()

        def quantize(off):
            peer = lax.rem(my + off, N_DEV)
            b = x_ref[pl.ds(peer * blk, blk), :]
            absmax = jnp.maximum(jnp.max(jnp.abs(b)), 1e-30)
            sc_tile[off - 1, :, :] = jnp.full((8, 128), absmax / 127.0,
                                              jnp.float32)
            xq[pl.ds(peer * blk, blk), :] = jnp.round(
                b * (127.0 / absmax)
            ).astype(jnp.int8)

        def send(off):
            peer = lax.rem(my + off, N_DEV)
            sc_rdma = pltpu.make_async_remote_copy(
                src_ref=sc_tile.at[off - 1],
                dst_ref=sc_recv.at[off - 1],
                send_sem=sc_send_sems.at[off - 1],
                recv_sem=sc_recv_sems.at[off - 1],
                device_id=(peer,),
                device_id_type=pl.DeviceIdType.MESH,
            )
            sc_rdma.start()
            rdma = pltpu.make_async_remote_copy(
                src_ref=xq.at[pl.ds(peer * blk, blk), :],
                dst_ref=recvq.at[off - 1],
                send_sem=send_sems.at[off - 1],
                recv_sem=recv_sems.at[off - 1],
                device_id=(peer,),
                device_id_type=pl.DeviceIdType.MESH,
            )
            rdma.start()
            return [sc_rdma, rdma]

        quantize(1)
        pl.semaphore_wait(barrier, N_DEV - 1)
        sends = send(1)
        quantize(3)
        sends += send(3)
        quantize(2)
        sends += send(2)

        xloc = x_ref[pl.ds(my * blk, blk), :]
        wdma0_half(0).wait()
        acc_a = jnp.dot(
            xloc, wf32[0, :, 0:h], preferred_element_type=jnp.float32
        )
        wdma0_half(1).wait()
        acc_b = jnp.dot(
            xloc, wf32[0, :, h:], preferred_element_type=jnp.float32
        )
        wdma(3).start()

        def wait_and_dequant(slot):
            src = lax.rem(my - (slot + 1) + N_DEV, N_DEV)
            sc_wait = pltpu.make_async_remote_copy(
                src_ref=sc_recv.at[slot],
                dst_ref=sc_recv.at[slot],
                send_sem=sc_send_sems.at[slot],
                recv_sem=sc_recv_sems.at[slot],
                device_id=(src,),
                device_id_type=pl.DeviceIdType.MESH,
            )
            sc_wait.wait_recv()
            recv = pltpu.make_async_remote_copy(
                src_ref=recvq.at[slot],
                dst_ref=recvq.at[slot],
                send_sem=send_sems.at[slot],
                recv_sem=recv_sems.at[slot],
                device_id=(src,),
                device_id_type=pl.DeviceIdType.MESH,
            )
            recv.wait_recv()
            return recvq[slot].astype(jnp.float32) * sc_recv[slot, 0, 0]

        for j, slot in ((1, 0), (2, 2)):
            wdma(j).wait()
            xhat = wait_and_dequant(slot)
            acc_a = acc_a + jnp.dot(
                xhat, wf32[bufs[j], :, 0:h],
                preferred_element_type=jnp.float32,
            )
            acc_b = acc_b + jnp.dot(
                xhat, wf32[bufs[j], :, h:],
                preferred_element_type=jnp.float32,
            )

        def gelu(v):
            c = 0.7978845608028654
            return 0.5 * v * (1.0 + jnp.tanh(c * (v + 0.044715 * v * v * v)))

        wdma(3).wait()
        xhat = wait_and_dequant(1)
        out_ref[:, 0:h] = gelu(
            acc_a + jnp.dot(
                xhat, wf32[0, :, 0:h], preferred_element_type=jnp.float32
            )
        )
        out_ref[:, h:] = gelu(
            acc_b + jnp.dot(
                xhat, wf32[0, :, h:], preferred_element_type=jnp.float32
            )
        )

        for rdma in sends:
            rdma.wait_send()

    return pl.pallas_call(
        body,
        out_shape=jax.ShapeDtypeStruct((blk, n), jnp.float32),
        in_specs=[
            pl.BlockSpec(memory_space=pltpu.VMEM),
            pl.BlockSpec(memory_space=pl.ANY),
        ],
        out_specs=pl.BlockSpec(memory_space=pltpu.VMEM),
        scratch_shapes=[
            pltpu.VMEM((m, k_per), jnp.int8),
            pltpu.VMEM((N_DEV - 1, 8, 128), jnp.float32),
            pltpu.VMEM((N_DEV - 1, blk, k_per), jnp.int8),
            pltpu.VMEM((N_DEV - 1, 8, 128), jnp.float32),
            pltpu.VMEM((3, blk, n), jnp.float32),
            pltpu.SemaphoreType.DMA((N_DEV - 1,)),
            pltpu.SemaphoreType.DMA((N_DEV - 1,)),
            pltpu.SemaphoreType.DMA((N_DEV - 1,)),
            pltpu.SemaphoreType.DMA((N_DEV - 1,)),
            pltpu.SemaphoreType.DMA((N_DEV + 1,)),
        ],
        compiler_params=pltpu.CompilerParams(collective_id=0),
    )(x, w_mat)
